# baseline (speedup 1.0000x reference)
import numpy as np
import jax
import jax.numpy as jnp
from jax.experimental import pallas as pl
from jax.experimental.pallas import tpu as pltpu

_C, _T, _H, _W = 3, 32, 224, 224
_TS = _T // 4
_D = _H * _W
_IDX = tuple(int(v) for v in np.linspace(0.0, _T - 1, _TS))
_ROWS = _C * _TS


def _gather_body(in_ref, out_ref, sem):
    descs = []
    for r in range(_ROWS):
        c, s = divmod(r, _TS)
        descs.append(pltpu.make_async_copy(
            in_ref.at[c * _T + _IDX[s]], out_ref.at[r], sem))
    for d in descs:
        d.start()
    for d in descs:
        d.wait()


_slow_gather = pl.pallas_call(
    _gather_body,
    out_shape=jax.ShapeDtypeStruct((_ROWS, _D), jnp.float32),
    in_specs=[pl.BlockSpec(memory_space=pl.ANY)],
    out_specs=pl.BlockSpec(memory_space=pl.ANY),
    scratch_shapes=[pltpu.SemaphoreType.DMA],
)


def kernel(frames):
    flat = frames.reshape(_C * _T, _D)
    slow = _slow_gather(flat).reshape(_C, _TS, _H, _W)
    return (slow, frames)


# speedup vs baseline: 2.8718x; 2.8718x over previous
import numpy as np
import jax
import jax.numpy as jnp
from jax.experimental import pallas as pl
from jax.experimental.pallas import tpu as pltpu

_C, _T, _H, _W = 3, 32, 224, 224
_TS = _T // 4
_D = _H * _W
_IDX = tuple(int(v) for v in np.linspace(0.0, _T - 1, _TS))
_ROWS = _C * _TS


def _src_row(r):
    c = r // _TS
    s = r % _TS
    t = 0
    for k, v in enumerate(_IDX):
        t = t + v * (s == k)
    return c * _T + t


def _gather_body(in_ref, out_ref):
    out_ref[...] = in_ref[...]


_SL = _D // 128

_slow_gather = pl.pallas_call(
    _gather_body,
    grid=(_ROWS,),
    out_shape=jax.ShapeDtypeStruct((_ROWS, _SL, 128), jnp.float32),
    in_specs=[pl.BlockSpec((1, _SL, 128), lambda r: (_src_row(r), 0, 0))],
    out_specs=pl.BlockSpec((1, _SL, 128), lambda r: (r, 0, 0)),
)


def kernel(frames):
    flat = frames.reshape(_C * _T, _SL, 128)
    slow = _slow_gather(flat).reshape(_C, _TS, _H, _W)
    return (slow, frames)


# speedup vs baseline: 3.1839x; 1.1087x over previous
import numpy as np
import jax
import jax.numpy as jnp
from jax.experimental import pallas as pl

_C, _T, _H, _W = 3, 32, 224, 224
_TS = _T // 4
_IDX = tuple(int(v) for v in np.linspace(0.0, _T - 1, _TS))


def _slot(t):
    s = -1
    for v in _IDX:
        s = s + (t >= v)
    return s


def _body(in_ref, fast_ref, slow_ref):
    x = in_ref[...]
    fast_ref[...] = x
    t = pl.program_id(1)
    sel = False
    for v in _IDX:
        sel = sel | (t == v)

    @pl.when(sel)
    def _():
        slow_ref[...] = x


_pack = pl.pallas_call(
    _body,
    grid=(_C, _T),
    out_shape=(
        jax.ShapeDtypeStruct((_C, _T, _H, _W), jnp.float32),
        jax.ShapeDtypeStruct((_C, _TS, _H, _W), jnp.float32),
    ),
    in_specs=[pl.BlockSpec((1, 1, _H, _W), lambda c, t: (c, t, 0, 0))],
    out_specs=(
        pl.BlockSpec((1, 1, _H, _W), lambda c, t: (c, t, 0, 0)),
        pl.BlockSpec((1, 1, _H, _W), lambda c, t: (c, _slot(t), 0, 0)),
    ),
)


def kernel(frames):
    fast, slow = _pack(frames)
    return (slow, fast)


# speedup vs baseline: 12.4558x; 3.9122x over previous
import numpy as np
import jax
import jax.numpy as jnp
from jax.experimental import pallas as pl

_C, _T, _H, _W = 3, 32, 224, 224
_TS = _T // 4
_IDX = tuple(int(v) for v in np.linspace(0.0, _T - 1, _TS))


def _body(in_ref, fast_ref, slow_ref):
    x = in_ref[...]
    fast_ref[...] = x
    for k, v in enumerate(_IDX):
        slow_ref[:, k] = x[:, v]


_pack = pl.pallas_call(
    _body,
    grid=(_C,),
    out_shape=(
        jax.ShapeDtypeStruct((_C, _T, _H, _W), jnp.float32),
        jax.ShapeDtypeStruct((_C, _TS, _H, _W), jnp.float32),
    ),
    in_specs=[pl.BlockSpec((1, _T, _H, _W), lambda c: (c, 0, 0, 0))],
    out_specs=(
        pl.BlockSpec((1, _T, _H, _W), lambda c: (c, 0, 0, 0)),
        pl.BlockSpec((1, _TS, _H, _W), lambda c: (c, 0, 0, 0)),
    ),
)


def kernel(frames):
    fast, slow = _pack(frames)
    return (slow, fast)
